# Initial kernel scaffold; baseline (speedup 1.0000x reference)
import jax
import jax.numpy as jnp
from jax.experimental import pallas as pl


def kernel(x, params):
    raise NotImplementedError("write your pallas kernel here")


# speedup vs baseline: 108.9151x; 108.9151x over previous
import functools

import jax
import jax.numpy as jnp
from jax import lax
from jax.experimental import pallas as pl
from jax.experimental.pallas import tpu as pltpu
from jax.experimental.pallas import tpu_sc as plsc

_B, _N, _K = 4, 4096, 20
_BN = _B * _N
_KBN = _K * _BN
_EPS = 1e-5
_KP = 32
_RT = 128
_RN = 512
_RF = 512
_NEG = -3.0e38


def _knn_body(xt_ref, xa_ref, idx_ref):
    b = pl.program_id(0)
    xt = xt_ref[0][:, :64]
    xa = xa_ref[0][:, :64]
    s = 2.0 * lax.dot_general(xt, xa, (((1,), (1,)), ((), ())),
                              preferred_element_type=jnp.float32)
    rn = jnp.sum(xt * xt, axis=1, keepdims=True)
    cn = jnp.sum(xa * xa, axis=1).reshape(1, _N)
    s = s - rn - cn
    colidx = lax.broadcasted_iota(jnp.int32, (_RT, _N), 1)
    ki = lax.broadcasted_iota(jnp.int32, (_RT, _KP), 1)
    base = b * _N
    acc = jnp.zeros((_RT, _KP), jnp.int32)
    for k in range(_K):
        m = jnp.max(s, axis=1, keepdims=True)
        cand = jnp.where(s >= m, colidx, _N)
        amin = jnp.min(cand, axis=1, keepdims=True)
        acc = jnp.where(ki == k, amin + base, acc)
        s = jnp.where(colidx == amin, _NEG, s)
    idx_ref[0] = acc


def _knn(h_table):
    return pl.pallas_call(
        _knn_body,
        grid=(_B, _N // _RT),
        in_specs=[
            pl.BlockSpec((1, _RT, 128), lambda b, t: (b, t, 0)),
            pl.BlockSpec((1, _N, 128), lambda b, t: (b, 0, 0)),
        ],
        out_specs=pl.BlockSpec((1, _RT, _KP), lambda b, t: (b, t, 0)),
        out_shape=jax.ShapeDtypeStruct((_B, _N, _KP), jnp.int32),
    )(h_table, h_table)


_SC_NW = 32
_SC_CH = 128


def _gather_rows(table, idxg):
    total = idxg.shape[0]
    per_w = total // _SC_NW
    nch = per_w // _SC_CH
    mesh = plsc.VectorSubcoreMesh(core_axis_name="c", subcore_axis_name="s")

    @functools.partial(
        pl.kernel,
        mesh=mesh,
        out_type=jax.ShapeDtypeStruct((total, 128), jnp.float32),
        scratch_types=[
            pltpu.VMEM((_SC_CH,), jnp.int32),
            pltpu.VMEM((_SC_CH, 128), jnp.float32),
            pltpu.SemaphoreType.DMA,
        ],
    )
    def k(table_hbm, idx_hbm, out_hbm, idx_v, rows_v, sem):
        wid = lax.axis_index("s") * 2 + lax.axis_index("c")
        base = wid * per_w

        def body(i, carry):
            off = base + i * _SC_CH
            pltpu.sync_copy(idx_hbm.at[pl.ds(off, _SC_CH)], idx_v)
            pltpu.async_copy(table_hbm.at[idx_v], rows_v, sem).wait()
            pltpu.sync_copy(rows_v, out_hbm.at[pl.ds(off, _SC_CH)])
            return carry

        lax.fori_loop(0, nch, body, 0)

    return k(table, idxg)


def _acc_stats(st_ref, y):
    s0 = jnp.sum(y, axis=0)
    s1 = jnp.sum(y * y, axis=0)
    y0 = s0 - st_ref[2, :]
    t0 = st_ref[0, :] + y0
    st_ref[2, :] = (t0 - st_ref[0, :]) - y0
    st_ref[0, :] = t0
    y1 = s1 - st_ref[3, :]
    t1 = st_ref[1, :] + y1
    st_ref[3, :] = (t1 - st_ref[1, :]) - y1
    st_ref[1, :] = t1


def _edge_feat(g, hc):
    lane = lax.broadcasted_iota(jnp.int32, g.shape, 1)
    return jnp.where(lane < 64, g - hc, hc)


def _pair_stats_body(g_ref, hc_ref, w1_ref, out_ref):
    step = pl.program_id(0) * pl.num_programs(1) + pl.program_id(1)

    @pl.when(step == 0)
    def _():
        out_ref[...] = jnp.zeros_like(out_ref)

    ef = _edge_feat(g_ref[...], hc_ref[...])
    y = jnp.dot(ef, w1_ref[...], preferred_element_type=jnp.float32)
    _acc_stats(out_ref, y)


def _pair_stats(g, table_rows, w1eT):
    nj = _BN // _RN
    return pl.pallas_call(
        _pair_stats_body,
        grid=(_K, nj),
        in_specs=[
            pl.BlockSpec((_RN, 128), lambda k, j: (k * nj + j, 0)),
            pl.BlockSpec((_RN, 128), lambda k, j: (j, 0)),
            pl.BlockSpec((128, 64), lambda k, j: (0, 0)),
        ],
        out_specs=pl.BlockSpec((8, 64), lambda k, j: (0, 0)),
        out_shape=jax.ShapeDtypeStruct((8, 64), jnp.float32),
    )(g, table_rows, w1eT)


def _layer2_body(g_ref, hc_ref, w1_ref, st_ref, w2_ref, hmax_ref, st2_ref):
    j = pl.program_id(0)
    k = pl.program_id(1)
    np_ = jnp.float32(_KBN)
    mean = st_ref[0, :] / np_
    var = st_ref[1, :] / np_ - mean * mean
    sd = jnp.sqrt(var + _EPS)
    ef = _edge_feat(g_ref[...], hc_ref[...])
    y1 = jnp.dot(ef, w1_ref[...], preferred_element_type=jnp.float32)
    v = (y1 - mean[None, :]) / sd[None, :]
    v = jnp.where(v > 0, v, 0.2 * v)
    y = jnp.dot(v, w2_ref[...], preferred_element_type=jnp.float32)

    @pl.when((j == 0) & (k == 0))
    def _():
        st2_ref[...] = jnp.zeros_like(st2_ref)

    _acc_stats(st2_ref, y)

    @pl.when(k == 0)
    def _():
        hmax_ref[...] = y

    @pl.when(k > 0)
    def _():
        hmax_ref[...] = jnp.maximum(hmax_ref[...], y)


def _layer2_max(g, table_rows, w1eT, stats1, w2T):
    nj = _BN // _RN
    return pl.pallas_call(
        _layer2_body,
        grid=(nj, _K),
        in_specs=[
            pl.BlockSpec((_RN, 128), lambda j, k: (k * nj + j, 0)),
            pl.BlockSpec((_RN, 128), lambda j, k: (j, 0)),
            pl.BlockSpec((128, 64), lambda j, k: (0, 0)),
            pl.BlockSpec((8, 64), lambda j, k: (0, 0)),
            pl.BlockSpec((64, 64), lambda j, k: (0, 0)),
        ],
        out_specs=[
            pl.BlockSpec((_RN, 64), lambda j, k: (j, 0)),
            pl.BlockSpec((8, 64), lambda j, k: (0, 0)),
        ],
        out_shape=[
            jax.ShapeDtypeStruct((_BN, 64), jnp.float32),
            jax.ShapeDtypeStruct((8, 64), jnp.float32),
        ],
    )(g, table_rows, w1eT, stats1, w2T)


def _stats_max_body(g_ref, hc_ref, w1_ref, st_ref, hmax_ref):
    j = pl.program_id(0)
    k = pl.program_id(1)
    ef = _edge_feat(g_ref[...], hc_ref[...])
    y = jnp.dot(ef, w1_ref[...], preferred_element_type=jnp.float32)

    @pl.when((j == 0) & (k == 0))
    def _():
        st_ref[...] = jnp.zeros_like(st_ref)

    _acc_stats(st_ref, y)

    @pl.when(k == 0)
    def _():
        hmax_ref[...] = y

    @pl.when(k > 0)
    def _():
        hmax_ref[...] = jnp.maximum(hmax_ref[...], y)


def _stats_max(g, table_rows, w1eT):
    nj = _BN // _RN
    return pl.pallas_call(
        _stats_max_body,
        grid=(nj, _K),
        in_specs=[
            pl.BlockSpec((_RN, 128), lambda j, k: (k * nj + j, 0)),
            pl.BlockSpec((_RN, 128), lambda j, k: (j, 0)),
            pl.BlockSpec((128, 64), lambda j, k: (0, 0)),
        ],
        out_specs=[
            pl.BlockSpec((8, 64), lambda j, k: (0, 0)),
            pl.BlockSpec((_RN, 64), lambda j, k: (j, 0)),
        ],
        out_shape=[
            jax.ShapeDtypeStruct((8, 64), jnp.float32),
            jax.ShapeDtypeStruct((_BN, 64), jnp.float32),
        ],
    )(g, table_rows, w1eT)


def _bn_lrelu(h, st_ref, np_):
    mean = st_ref[0, :] / np_
    var = st_ref[1, :] / np_ - mean * mean
    sd = jnp.sqrt(var + _EPS)
    v = (h - mean[None, :]) / sd[None, :]
    return jnp.where(v > 0, v, 0.2 * v)


def _finalize_body(h_ref, st_ref, out_ref, tab_ref):
    v = _bn_lrelu(h_ref[...], st_ref, jnp.float32(_KBN))
    out_ref[...] = v
    tab_ref[...] = jnp.concatenate([v, v], axis=1)


def _finalize_t_body(h_ref, st_ref, out_ref, tab_ref, outt_ref):
    v = _bn_lrelu(h_ref[...], st_ref, jnp.float32(_KBN))
    out_ref[...] = v
    tab_ref[...] = jnp.concatenate([v, v], axis=1)
    outt_ref[0] = v.T


def _finalize(hmax, stats):
    nj = _BN // _RF
    return pl.pallas_call(
        _finalize_body,
        grid=(nj,),
        in_specs=[
            pl.BlockSpec((_RF, 64), lambda j: (j, 0)),
            pl.BlockSpec((8, 64), lambda j: (0, 0)),
        ],
        out_specs=[
            pl.BlockSpec((_RF, 64), lambda j: (j, 0)),
            pl.BlockSpec((_RF, 128), lambda j: (j, 0)),
        ],
        out_shape=[
            jax.ShapeDtypeStruct((_BN, 64), jnp.float32),
            jax.ShapeDtypeStruct((_BN, 128), jnp.float32),
        ],
    )(hmax, stats)


def _finalize_t(hmax, stats):
    nj = _BN // _RF
    npb = _N // _RF
    return pl.pallas_call(
        _finalize_t_body,
        grid=(nj,),
        in_specs=[
            pl.BlockSpec((_RF, 64), lambda j: (j, 0)),
            pl.BlockSpec((8, 64), lambda j: (0, 0)),
        ],
        out_specs=[
            pl.BlockSpec((_RF, 64), lambda j: (j, 0)),
            pl.BlockSpec((_RF, 128), lambda j: (j, 0)),
            pl.BlockSpec((1, 64, _RF), lambda j: (j // npb, 0, j % npb)),
        ],
        out_shape=[
            jax.ShapeDtypeStruct((_BN, 64), jnp.float32),
            jax.ShapeDtypeStruct((_BN, 128), jnp.float32),
            jax.ShapeDtypeStruct((_B, 64, _N), jnp.float32),
        ],
    )(hmax, stats)


def _mlp1_body(x_ref, w_ref, y_ref, st_ref):
    y = jnp.dot(x_ref[...], w_ref[...], preferred_element_type=jnp.float32)

    @pl.when(pl.program_id(0) == 0)
    def _():
        st_ref[...] = jnp.zeros_like(st_ref)

    _acc_stats(st_ref, y)
    y_ref[...] = y


def _mlp1(x, wT):
    din, dout = wT.shape
    nj = _BN // _RF
    return pl.pallas_call(
        _mlp1_body,
        grid=(nj,),
        in_specs=[
            pl.BlockSpec((_RF, din), lambda j: (j, 0)),
            pl.BlockSpec((din, dout), lambda j: (0, 0)),
        ],
        out_specs=[
            pl.BlockSpec((_RF, dout), lambda j: (j, 0)),
            pl.BlockSpec((8, dout), lambda j: (0, 0)),
        ],
        out_shape=[
            jax.ShapeDtypeStruct((_BN, dout), jnp.float32),
            jax.ShapeDtypeStruct((8, dout), jnp.float32),
        ],
    )(x, wT)


def _mlp2_body(x_ref, st_ref, w_ref, y_ref, st2_ref):
    v = _bn_lrelu(x_ref[...], st_ref, jnp.float32(_BN))
    y = jnp.dot(v, w_ref[...], preferred_element_type=jnp.float32)

    @pl.when(pl.program_id(0) == 0)
    def _():
        st2_ref[...] = jnp.zeros_like(st2_ref)

    _acc_stats(st2_ref, y)
    y_ref[...] = y


def _mlp2(x, stats1, wT):
    din, dout = wT.shape
    nj = _BN // _RF
    return pl.pallas_call(
        _mlp2_body,
        grid=(nj,),
        in_specs=[
            pl.BlockSpec((_RF, din), lambda j: (j, 0)),
            pl.BlockSpec((8, din), lambda j: (0, 0)),
            pl.BlockSpec((din, dout), lambda j: (0, 0)),
        ],
        out_specs=[
            pl.BlockSpec((_RF, dout), lambda j: (j, 0)),
            pl.BlockSpec((8, dout), lambda j: (0, 0)),
        ],
        out_shape=[
            jax.ShapeDtypeStruct((_BN, dout), jnp.float32),
            jax.ShapeDtypeStruct((8, dout), jnp.float32),
        ],
    )(x, stats1, wT)


def _mlp_fin_body(x_ref, st_ref, out_ref):
    v = _bn_lrelu(x_ref[...], st_ref, jnp.float32(_BN))
    out_ref[0] = v.T


def _mlp_fin(x, stats):
    dout = x.shape[-1]
    nj = _BN // _RF
    npb = _N // _RF
    return pl.pallas_call(
        _mlp_fin_body,
        grid=(nj,),
        in_specs=[
            pl.BlockSpec((_RF, dout), lambda j: (j, 0)),
            pl.BlockSpec((8, dout), lambda j: (0, 0)),
        ],
        out_specs=pl.BlockSpec((1, dout, _RF), lambda j: (j // npb, 0, j % npb)),
        out_shape=jax.ShapeDtypeStruct((_B, dout, _N), jnp.float32),
    )(x, stats)


def _prep_w1(w, c):
    wt = jnp.zeros((128, 64), jnp.float32)
    wt = wt.at[:c, :].set(jnp.transpose(w[:, :c]))
    wt = wt.at[64:64 + c, :].set(jnp.transpose(w[:, c:]))
    return wt


def _edge_block(h_table, layers, c, want_transposed):
    idx = _knn(h_table)
    idx_t = jnp.transpose(idx[:, :, :_K].reshape(_BN, _K)).reshape(_KBN)
    table_rows = h_table.reshape(_BN, 128)
    g = _gather_rows(table_rows, idx_t)
    w1eT = _prep_w1(layers[0]['W'], c)
    if len(layers) == 2:
        stats1 = _pair_stats(g, table_rows, w1eT)
        w2T = jnp.transpose(layers[1]['W'])
        hmax, stats2 = _layer2_max(g, table_rows, w1eT, stats1, w2T)
    else:
        stats2, hmax = _stats_max(g, table_rows, w1eT)
    if want_transposed:
        return tuple(_finalize_t(hmax, stats2))
    return tuple(_finalize(hmax, stats2)) + (None,)


def kernel(x, params):
    xr = jnp.transpose(x, (0, 2, 1))
    zp = jnp.zeros((_B, _N, 61), jnp.float32)
    table0 = jnp.concatenate([xr, zp, xr, zp], axis=-1)
    blocks = params['edge']

    h1, t1, out0 = _edge_block(table0, blocks[0], 3, True)
    h2, t2, _ = _edge_block(t1.reshape(_B, _N, 128), blocks[1], 64, False)
    h3, _, _ = _edge_block(t2.reshape(_B, _N, 128), blocks[2], 64, False)

    cat = jnp.concatenate([h1, h2, h3], axis=-1)
    y1, st1 = _mlp1(cat, jnp.transpose(params['mlp'][0]['W']))
    y2, st2 = _mlp2(y1, st1, jnp.transpose(params['mlp'][1]['W']))
    out = _mlp_fin(y2, st2)
    return (out0, out)


# speedup vs baseline: 126.9448x; 1.1655x over previous
import functools

import jax
import jax.numpy as jnp
from jax import lax
from jax.experimental import pallas as pl
from jax.experimental.pallas import tpu as pltpu
from jax.experimental.pallas import tpu_sc as plsc

_B, _N, _K = 4, 4096, 20
_BN = _B * _N
_KBN = _K * _BN
_EPS = 1e-5
_KP = 32
_RT = 128
_RN = 1024
_RF = 512
_NEG = -3.0e38


def _knn_body(xt_ref, xa_ref, idx_ref):
    b = pl.program_id(0)
    xt = xt_ref[0][:, :64]
    xa = xa_ref[0][:, :64]
    s = 2.0 * lax.dot_general(xt, xa, (((1,), (1,)), ((), ())),
                              preferred_element_type=jnp.float32)
    rn = jnp.sum(xt * xt, axis=1, keepdims=True)
    cn = jnp.sum(xa * xa, axis=1).reshape(1, _N)
    s = s - rn - cn
    colidx = lax.broadcasted_iota(jnp.int32, (_RT, _N), 1)
    ki = lax.broadcasted_iota(jnp.int32, (_RT, _KP), 1)
    base = b * _N
    acc = jnp.zeros((_RT, _KP), jnp.int32)
    for k in range(_K):
        m = jnp.max(s, axis=1, keepdims=True)
        cand = jnp.where(s >= m, colidx, _N)
        amin = jnp.min(cand, axis=1, keepdims=True)
        acc = jnp.where(ki == k, amin + base, acc)
        s = jnp.where(colidx == amin, _NEG, s)
    idx_ref[0] = acc


def _knn(h_table):
    return pl.pallas_call(
        _knn_body,
        grid=(_B, _N // _RT),
        in_specs=[
            pl.BlockSpec((1, _RT, 128), lambda b, t: (b, t, 0)),
            pl.BlockSpec((1, _N, 128), lambda b, t: (b, 0, 0)),
        ],
        out_specs=pl.BlockSpec((1, _RT, _KP), lambda b, t: (b, t, 0)),
        out_shape=jax.ShapeDtypeStruct((_B, _N, _KP), jnp.int32),
    )(h_table, h_table)


_SC_NW = 32
_SC_CH = 512


def _gather_rows(table, idxg):
    total = idxg.shape[0]
    per_w = total // _SC_NW
    nch = per_w // _SC_CH
    mesh = plsc.VectorSubcoreMesh(core_axis_name="c", subcore_axis_name="s")

    @functools.partial(
        pl.kernel,
        mesh=mesh,
        out_type=jax.ShapeDtypeStruct((total, 128), jnp.float32),
        scratch_types=[
            pltpu.VMEM((_SC_CH,), jnp.int32),
            pltpu.VMEM((_SC_CH, 128), jnp.float32),
            pltpu.SemaphoreType.DMA,
        ],
    )
    def k(table_hbm, idx_hbm, out_hbm, idx_v, rows_v, sem):
        wid = lax.axis_index("s") * 2 + lax.axis_index("c")
        base = wid * per_w

        def body(i, carry):
            off = base + i * _SC_CH
            pltpu.sync_copy(idx_hbm.at[pl.ds(off, _SC_CH)], idx_v)
            pltpu.async_copy(table_hbm.at[idx_v], rows_v, sem).wait()
            pltpu.sync_copy(rows_v, out_hbm.at[pl.ds(off, _SC_CH)])
            return carry

        lax.fori_loop(0, nch, body, 0)

    return k(table, idxg)


def _acc_stats(st_ref, y):
    s0 = jnp.sum(y, axis=0)
    s1 = jnp.sum(y * y, axis=0)
    y0 = s0 - st_ref[2, :]
    t0 = st_ref[0, :] + y0
    st_ref[2, :] = (t0 - st_ref[0, :]) - y0
    st_ref[0, :] = t0
    y1 = s1 - st_ref[3, :]
    t1 = st_ref[1, :] + y1
    st_ref[3, :] = (t1 - st_ref[1, :]) - y1
    st_ref[1, :] = t1


def _edge_feat(g, hc):
    lane = lax.broadcasted_iota(jnp.int32, g.shape, 1)
    return jnp.where(lane < 64, g - hc, hc)


def _pair_stats_body(g_ref, hc_ref, w1_ref, out_ref):
    step = pl.program_id(0) * pl.num_programs(1) + pl.program_id(1)

    @pl.when(step == 0)
    def _():
        out_ref[...] = jnp.zeros_like(out_ref)

    ef = _edge_feat(g_ref[...], hc_ref[...])
    y = jnp.dot(ef, w1_ref[...], preferred_element_type=jnp.float32)
    _acc_stats(out_ref, y)


def _pair_stats(g, table_rows, w1eT):
    nj = _BN // _RN
    return pl.pallas_call(
        _pair_stats_body,
        grid=(_K, nj),
        in_specs=[
            pl.BlockSpec((_RN, 128), lambda k, j: (k * nj + j, 0)),
            pl.BlockSpec((_RN, 128), lambda k, j: (j, 0)),
            pl.BlockSpec((128, 64), lambda k, j: (0, 0)),
        ],
        out_specs=pl.BlockSpec((8, 64), lambda k, j: (0, 0)),
        out_shape=jax.ShapeDtypeStruct((8, 64), jnp.float32),
    )(g, table_rows, w1eT)


def _layer2_body(g_ref, hc_ref, w1_ref, st_ref, w2_ref, hmax_ref, st2_ref):
    j = pl.program_id(0)
    k = pl.program_id(1)
    np_ = jnp.float32(_KBN)
    mean = st_ref[0, :] / np_
    var = st_ref[1, :] / np_ - mean * mean
    sd = jnp.sqrt(var + _EPS)
    ef = _edge_feat(g_ref[...], hc_ref[...])
    y1 = jnp.dot(ef, w1_ref[...], preferred_element_type=jnp.float32)
    v = (y1 - mean[None, :]) / sd[None, :]
    v = jnp.where(v > 0, v, 0.2 * v)
    y = jnp.dot(v, w2_ref[...], preferred_element_type=jnp.float32)

    @pl.when((j == 0) & (k == 0))
    def _():
        st2_ref[...] = jnp.zeros_like(st2_ref)

    _acc_stats(st2_ref, y)

    @pl.when(k == 0)
    def _():
        hmax_ref[...] = y

    @pl.when(k > 0)
    def _():
        hmax_ref[...] = jnp.maximum(hmax_ref[...], y)


def _layer2_max(g, table_rows, w1eT, stats1, w2T):
    nj = _BN // _RN
    return pl.pallas_call(
        _layer2_body,
        grid=(nj, _K),
        in_specs=[
            pl.BlockSpec((_RN, 128), lambda j, k: (k * nj + j, 0)),
            pl.BlockSpec((_RN, 128), lambda j, k: (j, 0)),
            pl.BlockSpec((128, 64), lambda j, k: (0, 0)),
            pl.BlockSpec((8, 64), lambda j, k: (0, 0)),
            pl.BlockSpec((64, 64), lambda j, k: (0, 0)),
        ],
        out_specs=[
            pl.BlockSpec((_RN, 64), lambda j, k: (j, 0)),
            pl.BlockSpec((8, 64), lambda j, k: (0, 0)),
        ],
        out_shape=[
            jax.ShapeDtypeStruct((_BN, 64), jnp.float32),
            jax.ShapeDtypeStruct((8, 64), jnp.float32),
        ],
    )(g, table_rows, w1eT, stats1, w2T)


def _stats_max_body(g_ref, hc_ref, w1_ref, st_ref, hmax_ref):
    j = pl.program_id(0)
    k = pl.program_id(1)
    ef = _edge_feat(g_ref[...], hc_ref[...])
    y = jnp.dot(ef, w1_ref[...], preferred_element_type=jnp.float32)

    @pl.when((j == 0) & (k == 0))
    def _():
        st_ref[...] = jnp.zeros_like(st_ref)

    _acc_stats(st_ref, y)

    @pl.when(k == 0)
    def _():
        hmax_ref[...] = y

    @pl.when(k > 0)
    def _():
        hmax_ref[...] = jnp.maximum(hmax_ref[...], y)


def _stats_max(g, table_rows, w1eT):
    nj = _BN // _RN
    return pl.pallas_call(
        _stats_max_body,
        grid=(nj, _K),
        in_specs=[
            pl.BlockSpec((_RN, 128), lambda j, k: (k * nj + j, 0)),
            pl.BlockSpec((_RN, 128), lambda j, k: (j, 0)),
            pl.BlockSpec((128, 64), lambda j, k: (0, 0)),
        ],
        out_specs=[
            pl.BlockSpec((8, 64), lambda j, k: (0, 0)),
            pl.BlockSpec((_RN, 64), lambda j, k: (j, 0)),
        ],
        out_shape=[
            jax.ShapeDtypeStruct((8, 64), jnp.float32),
            jax.ShapeDtypeStruct((_BN, 64), jnp.float32),
        ],
    )(g, table_rows, w1eT)


def _bn_lrelu(h, st_ref, np_):
    mean = st_ref[0, :] / np_
    var = st_ref[1, :] / np_ - mean * mean
    sd = jnp.sqrt(var + _EPS)
    v = (h - mean[None, :]) / sd[None, :]
    return jnp.where(v > 0, v, 0.2 * v)


def _finalize_body(h_ref, st_ref, out_ref, tab_ref):
    v = _bn_lrelu(h_ref[...], st_ref, jnp.float32(_KBN))
    out_ref[...] = v
    tab_ref[...] = jnp.concatenate([v, v], axis=1)


def _finalize_t_body(h_ref, st_ref, out_ref, tab_ref, outt_ref):
    v = _bn_lrelu(h_ref[...], st_ref, jnp.float32(_KBN))
    out_ref[...] = v
    tab_ref[...] = jnp.concatenate([v, v], axis=1)
    outt_ref[0] = v.T


def _finalize(hmax, stats):
    nj = _BN // _RF
    return pl.pallas_call(
        _finalize_body,
        grid=(nj,),
        in_specs=[
            pl.BlockSpec((_RF, 64), lambda j: (j, 0)),
            pl.BlockSpec((8, 64), lambda j: (0, 0)),
        ],
        out_specs=[
            pl.BlockSpec((_RF, 64), lambda j: (j, 0)),
            pl.BlockSpec((_RF, 128), lambda j: (j, 0)),
        ],
        out_shape=[
            jax.ShapeDtypeStruct((_BN, 64), jnp.float32),
            jax.ShapeDtypeStruct((_BN, 128), jnp.float32),
        ],
    )(hmax, stats)


def _finalize_t(hmax, stats):
    nj = _BN // _RF
    npb = _N // _RF
    return pl.pallas_call(
        _finalize_t_body,
        grid=(nj,),
        in_specs=[
            pl.BlockSpec((_RF, 64), lambda j: (j, 0)),
            pl.BlockSpec((8, 64), lambda j: (0, 0)),
        ],
        out_specs=[
            pl.BlockSpec((_RF, 64), lambda j: (j, 0)),
            pl.BlockSpec((_RF, 128), lambda j: (j, 0)),
            pl.BlockSpec((1, 64, _RF), lambda j: (j // npb, 0, j % npb)),
        ],
        out_shape=[
            jax.ShapeDtypeStruct((_BN, 64), jnp.float32),
            jax.ShapeDtypeStruct((_BN, 128), jnp.float32),
            jax.ShapeDtypeStruct((_B, 64, _N), jnp.float32),
        ],
    )(hmax, stats)


def _mlp1_body(x_ref, w_ref, y_ref, st_ref):
    y = jnp.dot(x_ref[...], w_ref[...], preferred_element_type=jnp.float32)

    @pl.when(pl.program_id(0) == 0)
    def _():
        st_ref[...] = jnp.zeros_like(st_ref)

    _acc_stats(st_ref, y)
    y_ref[...] = y


def _mlp1(x, wT):
    din, dout = wT.shape
    nj = _BN // _RF
    return pl.pallas_call(
        _mlp1_body,
        grid=(nj,),
        in_specs=[
            pl.BlockSpec((_RF, din), lambda j: (j, 0)),
            pl.BlockSpec((din, dout), lambda j: (0, 0)),
        ],
        out_specs=[
            pl.BlockSpec((_RF, dout), lambda j: (j, 0)),
            pl.BlockSpec((8, dout), lambda j: (0, 0)),
        ],
        out_shape=[
            jax.ShapeDtypeStruct((_BN, dout), jnp.float32),
            jax.ShapeDtypeStruct((8, dout), jnp.float32),
        ],
    )(x, wT)


def _mlp2_body(x_ref, st_ref, w_ref, y_ref, st2_ref):
    v = _bn_lrelu(x_ref[...], st_ref, jnp.float32(_BN))
    y = jnp.dot(v, w_ref[...], preferred_element_type=jnp.float32)

    @pl.when(pl.program_id(0) == 0)
    def _():
        st2_ref[...] = jnp.zeros_like(st2_ref)

    _acc_stats(st2_ref, y)
    y_ref[...] = y


def _mlp2(x, stats1, wT):
    din, dout = wT.shape
    nj = _BN // _RF
    return pl.pallas_call(
        _mlp2_body,
        grid=(nj,),
        in_specs=[
            pl.BlockSpec((_RF, din), lambda j: (j, 0)),
            pl.BlockSpec((8, din), lambda j: (0, 0)),
            pl.BlockSpec((din, dout), lambda j: (0, 0)),
        ],
        out_specs=[
            pl.BlockSpec((_RF, dout), lambda j: (j, 0)),
            pl.BlockSpec((8, dout), lambda j: (0, 0)),
        ],
        out_shape=[
            jax.ShapeDtypeStruct((_BN, dout), jnp.float32),
            jax.ShapeDtypeStruct((8, dout), jnp.float32),
        ],
    )(x, stats1, wT)


def _mlp_fin_body(x_ref, st_ref, out_ref):
    v = _bn_lrelu(x_ref[...], st_ref, jnp.float32(_BN))
    out_ref[0] = v.T


def _mlp_fin(x, stats):
    dout = x.shape[-1]
    nj = _BN // _RF
    npb = _N // _RF
    return pl.pallas_call(
        _mlp_fin_body,
        grid=(nj,),
        in_specs=[
            pl.BlockSpec((_RF, dout), lambda j: (j, 0)),
            pl.BlockSpec((8, dout), lambda j: (0, 0)),
        ],
        out_specs=pl.BlockSpec((1, dout, _RF), lambda j: (j // npb, 0, j % npb)),
        out_shape=jax.ShapeDtypeStruct((_B, dout, _N), jnp.float32),
    )(x, stats)


def _prep_w1(w, c):
    wt = jnp.zeros((128, 64), jnp.float32)
    wt = wt.at[:c, :].set(jnp.transpose(w[:, :c]))
    wt = wt.at[64:64 + c, :].set(jnp.transpose(w[:, c:]))
    return wt


def _edge_block(h_table, layers, c, want_transposed):
    idx = _knn(h_table)
    idx_t = jnp.transpose(idx[:, :, :_K].reshape(_BN, _K)).reshape(_KBN)
    table_rows = h_table.reshape(_BN, 128)
    g = _gather_rows(table_rows, idx_t)
    w1eT = _prep_w1(layers[0]['W'], c)
    if len(layers) == 2:
        stats1 = _pair_stats(g, table_rows, w1eT)
        w2T = jnp.transpose(layers[1]['W'])
        hmax, stats2 = _layer2_max(g, table_rows, w1eT, stats1, w2T)
    else:
        stats2, hmax = _stats_max(g, table_rows, w1eT)
    if want_transposed:
        return tuple(_finalize_t(hmax, stats2))
    return tuple(_finalize(hmax, stats2)) + (None,)


def kernel(x, params):
    xr = jnp.transpose(x, (0, 2, 1))
    zp = jnp.zeros((_B, _N, 61), jnp.float32)
    table0 = jnp.concatenate([xr, zp, xr, zp], axis=-1)
    blocks = params['edge']

    h1, t1, out0 = _edge_block(table0, blocks[0], 3, True)
    h2, t2, _ = _edge_block(t1.reshape(_B, _N, 128), blocks[1], 64, False)
    h3, _, _ = _edge_block(t2.reshape(_B, _N, 128), blocks[2], 64, False)

    cat = jnp.concatenate([h1, h2, h3], axis=-1)
    y1, st1 = _mlp1(cat, jnp.transpose(params['mlp'][0]['W']))
    y2, st2 = _mlp2(y1, st1, jnp.transpose(params['mlp'][1]['W']))
    out = _mlp_fin(y2, st2)
    return (out0, out)
